# Initial kernel scaffold; baseline (speedup 1.0000x reference)
import jax
import jax.numpy as jnp
from jax.experimental import pallas as pl


def kernel(x, pre_mask, pruning_index, ln_w, ln_b, W1, b1, W2, b2, W3, b3, W4, b4, gumbel):
    raise NotImplementedError("write your pallas kernel here")


# speedup vs baseline: 2.4309x; 2.4309x over previous
import functools

import jax
import jax.numpy as jnp
import numpy as np
from jax.experimental import pallas as pl
from jax.experimental.pallas import tpu as pltpu


def _gelu(v):
    return jnp.array(0.5, v.dtype) * v * (jax.lax.erf(v / np.float32(np.sqrt(2.0))) + 1.0)


def _mask_body(x_ref, pm_ref, lnw_ref, lnb_ref, W1_ref, b1_ref, W2_ref,
               b2_ref, W3_ref, b3_ref, W4_ref, b4_ref, g_ref, out_ref,
               h_scr, gsum, psum, gvec, *, chunk, n, c2):
    b = pl.program_id(0)
    p = pl.program_id(1)
    c = pl.program_id(2)

    @pl.when(p == 0)
    def _phase0():
        xb = x_ref[0, :, :]
        mu = jnp.mean(xb, axis=1, keepdims=True)
        var = jnp.mean((xb - mu) ** 2, axis=1, keepdims=True)
        xn = (xb - mu) / jnp.sqrt(var + 1e-5) * lnw_ref[0, :] + lnb_ref[0, :]
        h = _gelu(jnp.dot(xn, W1_ref[...]) + b1_ref[0, :])
        pm = pm_ref[0, :, :]
        h_scr[b, pl.ds(c * chunk, chunk), :] = h[:, :c2]
        contrib = jnp.sum(h[:, c2:] * pm, axis=0).reshape(1, c2)
        pc = jnp.sum(pm).reshape(1, 1)

        @pl.when(c == 0)
        def _():
            gsum[pl.ds(b, 1), :] = contrib
            psum[pl.ds(b, 1), 0:1] = pc

        @pl.when(c != 0)
        def _():
            gsum[pl.ds(b, 1), :] += contrib
            psum[pl.ds(b, 1), 0:1] += pc

    @pl.when(p == 1)
    def _phase1():
        @pl.when(c == 0)
        def _():
            gmean = gsum[pl.ds(b, 1), :] / psum[pl.ds(b, 1), 0:1]
            gvec[pl.ds(b, 1), :] = jnp.dot(gmean, W2_ref[c2:, :]) + b2_ref[0, :]

        h1 = h_scr[b, pl.ds(c * chunk, chunk), :]
        h2 = _gelu(jnp.dot(h1, W2_ref[:c2, :]) + gvec[pl.ds(b, 1), :])
        h3 = _gelu(jnp.dot(h2, W3_ref[...]) + b3_ref[0, :])
        z = jnp.dot(h3, W4_ref[...]) + b4_ref[0, :]
        g = g_ref[0, :, :]
        t = (z[:, 0:1] + g[:, 0:1]) - (z[:, 1:2] + g[:, 1:2])
        y = jnp.where(t >= 0, jnp.float32(1.0), jnp.float32(0.0))
        out_ref[0, :, :] = y * pm_ref[0, :, :]


def kernel(x, pre_mask, pruning_index, ln_w, ln_b, W1, b1, W2, b2, W3, b3,
           W4, b4, gumbel):
    del pruning_index
    N, B, C = x.shape
    c2 = C // 2
    c4 = C // 4
    chunk = 512
    nc = N // chunk

    body = functools.partial(_mask_body, chunk=chunk, n=N, c2=c2)
    out = pl.pallas_call(
        body,
        grid=(B, 2, nc),
        in_specs=[
            pl.BlockSpec((1, chunk, C), lambda b, p, c: (b, c * (1 - p), 0)),
            pl.BlockSpec((1, chunk, 1), lambda b, p, c: (b, c, 0)),
            pl.BlockSpec((1, C), lambda b, p, c: (0, 0)),
            pl.BlockSpec((1, C), lambda b, p, c: (0, 0)),
            pl.BlockSpec((C, C), lambda b, p, c: (0, 0)),
            pl.BlockSpec((1, C), lambda b, p, c: (0, 0)),
            pl.BlockSpec((C, c2), lambda b, p, c: (0, 0)),
            pl.BlockSpec((1, c2), lambda b, p, c: (0, 0)),
            pl.BlockSpec((c2, c4), lambda b, p, c: (0, 0)),
            pl.BlockSpec((1, c4), lambda b, p, c: (0, 0)),
            pl.BlockSpec((c4, 2), lambda b, p, c: (0, 0)),
            pl.BlockSpec((1, 2), lambda b, p, c: (0, 0)),
            pl.BlockSpec((1, chunk, 2), lambda b, p, c: (b, c, 0)),
        ],
        out_specs=pl.BlockSpec((1, chunk, 1), lambda b, p, c: (b, p * c, 0)),
        out_shape=jax.ShapeDtypeStruct((B, N, 1), jnp.float32),
        scratch_shapes=[
            pltpu.VMEM((B, N, c2), jnp.float32),
            pltpu.VMEM((8, c2), jnp.float32),
            pltpu.VMEM((8, 128), jnp.float32),
            pltpu.VMEM((8, c2), jnp.float32),
        ],
    )(jnp.transpose(x, (1, 0, 2)), pre_mask, ln_w.reshape(1, C), ln_b.reshape(1, C), W1,
      b1.reshape(1, C), W2, b2.reshape(1, c2), W3, b3.reshape(1, c4),
      W4, b4.reshape(1, 2), gumbel)
    return out


# speedup vs baseline: 3.1960x; 1.3147x over previous
import functools

import jax
import jax.numpy as jnp
import numpy as np
from jax.experimental import pallas as pl
from jax.experimental.pallas import tpu as pltpu


def _gelu(v):
    return jnp.array(0.5, v.dtype) * v * (jax.lax.erf(v / np.float32(np.sqrt(2.0))) + 1.0)


def _mask_body(x_ref, pm_ref, lnw_ref, lnb_ref, W1_ref, b1_ref, W2_ref,
               b2_ref, W3_ref, b3_ref, W4_ref, b4_ref, g_ref, out_ref,
               h_scr, gsum, psum, gvec, *, chunk, nbatch, c2):
    p = pl.program_id(0)
    c = pl.program_id(1)

    @pl.when(p == 0)
    def _phase0():
        for b in range(nbatch):
            xb = x_ref[:, b, :]
            mu = jnp.mean(xb, axis=1, keepdims=True)
            var = jnp.mean((xb - mu) ** 2, axis=1, keepdims=True)
            xn = (xb - mu) / jnp.sqrt(var + 1e-5) * lnw_ref[0, :] + lnb_ref[0, :]
            h = _gelu(jnp.dot(xn, W1_ref[...]) + b1_ref[0, :])
            pm = pm_ref[b, :, :]
            h_scr[b, pl.ds(c * chunk, chunk), :] = h[:, :c2]
            contrib = jnp.sum(h[:, c2:] * pm, axis=0).reshape(1, c2)
            pc = jnp.sum(pm).reshape(1, 1)

            @pl.when(c == 0)
            def _(b=b, contrib=contrib, pc=pc):
                gsum[pl.ds(b, 1), :] = contrib
                psum[pl.ds(b, 1), 0:1] = pc

            @pl.when(c != 0)
            def _(b=b, contrib=contrib, pc=pc):
                gsum[pl.ds(b, 1), :] += contrib
                psum[pl.ds(b, 1), 0:1] += pc

    @pl.when(p == 1)
    def _phase1():
        @pl.when(c == 0)
        def _():
            gmean = gsum[0:nbatch, :] / psum[0:nbatch, 0:1]
            gvec[0:nbatch, :] = jnp.dot(gmean, W2_ref[c2:, :]) + b2_ref[0, :]

        for b in range(nbatch):
            h1 = h_scr[b, pl.ds(c * chunk, chunk), :]
            h2 = _gelu(jnp.dot(h1, W2_ref[:c2, :]) + gvec[pl.ds(b, 1), :])
            h3 = _gelu(jnp.dot(h2, W3_ref[...]) + b3_ref[0, :])
            z = jnp.dot(h3, W4_ref[...]) + b4_ref[0, :]
            g = g_ref[b, :, :]
            t = (z[:, 0:1] + g[:, 0:1]) - (z[:, 1:2] + g[:, 1:2])
            y = jnp.where(t >= 0, jnp.float32(1.0), jnp.float32(0.0))
            out_ref[b, :, :] = y * pm_ref[b, :, :]


def kernel(x, pre_mask, pruning_index, ln_w, ln_b, W1, b1, W2, b2, W3, b3,
           W4, b4, gumbel):
    del pruning_index
    N, B, C = x.shape
    c2 = C // 2
    c4 = C // 4
    chunk = 512
    nc = N // chunk

    body = functools.partial(_mask_body, chunk=chunk, nbatch=B, c2=c2)
    out = pl.pallas_call(
        body,
        grid=(2, nc),
        in_specs=[
            pl.BlockSpec((chunk, B, C), lambda p, c: (c * (1 - p), 0, 0)),
            pl.BlockSpec((B, chunk, 1), lambda p, c: (0, c, 0)),
            pl.BlockSpec((1, C), lambda p, c: (0, 0)),
            pl.BlockSpec((1, C), lambda p, c: (0, 0)),
            pl.BlockSpec((C, C), lambda p, c: (0, 0)),
            pl.BlockSpec((1, C), lambda p, c: (0, 0)),
            pl.BlockSpec((C, c2), lambda p, c: (0, 0)),
            pl.BlockSpec((1, c2), lambda p, c: (0, 0)),
            pl.BlockSpec((c2, c4), lambda p, c: (0, 0)),
            pl.BlockSpec((1, c4), lambda p, c: (0, 0)),
            pl.BlockSpec((c4, 2), lambda p, c: (0, 0)),
            pl.BlockSpec((1, 2), lambda p, c: (0, 0)),
            pl.BlockSpec((B, chunk, 2), lambda p, c: (0, c, 0)),
        ],
        out_specs=pl.BlockSpec((B, chunk, 1), lambda p, c: (0, p * c, 0)),
        out_shape=jax.ShapeDtypeStruct((B, N, 1), jnp.float32),
        scratch_shapes=[
            pltpu.VMEM((B, N, c2), jnp.float32),
            pltpu.VMEM((8, c2), jnp.float32),
            pltpu.VMEM((8, 128), jnp.float32),
            pltpu.VMEM((8, c2), jnp.float32),
        ],
    )(x, pre_mask, ln_w.reshape(1, C), ln_b.reshape(1, C), W1,
      b1.reshape(1, C), W2, b2.reshape(1, c2), W3, b3.reshape(1, c4),
      W4, b4.reshape(1, 2), gumbel)
    return out
